# Initial kernel scaffold; baseline (speedup 1.0000x reference)
import functools

import jax
import jax.numpy as jnp
from jax.experimental import pallas as pl
from jax.experimental.pallas import tpu as pltpu
from jax.experimental.pallas import tpu_sc as plsc

_KS = 3
_N = _KS * _KS
_C = 96
_H = 224
_W = 224
_HP = _H + 2
_NPIX = _H * _W
_NSAMP = _N * _NPIX
_GWIN = 128
_NWORK = 32
_NSAMP_PAD = ((_NSAMP + _GWIN * _NWORK - 1) // (_GWIN * _NWORK)) * (_GWIN * _NWORK)
_T = 256
_NT = _NPIX // _T


def _idx_weights_body(offh_ref, offw_ref, idx_ref,
                      w00_ref, w01_ref, w10_ref, w11_ref):
    k = pl.program_id(0)
    dkx = (k // 3 - 1).astype(jnp.float32)
    dky = (k % 3 - 1).astype(jnp.float32)
    shape = offh_ref.shape
    hh = jax.lax.broadcasted_iota(jnp.float32, shape, 1)
    ww = jax.lax.broadcasted_iota(jnp.float32, shape, 2)
    hmax = jnp.float32(_HP - 1)
    ph = (hh + (1.0 + dkx)) + offh_ref[...]
    pw = (ww + (1.0 + dky)) + offw_ref[...]
    fh = jnp.floor(ph)
    fw = jnp.floor(pw)
    qlth = jnp.clip(fh, 0.0, hmax)
    qrbh = jnp.clip(fh + 1.0, 0.0, hmax)
    qltw = jnp.clip(fw, 0.0, hmax)
    qrbw = jnp.clip(fw + 1.0, 0.0, hmax)
    mh = jnp.logical_or(ph < 1.0, ph > hmax - 1.0)
    mw = jnp.logical_or(pw < 1.0, pw > hmax - 1.0)
    ph2 = jnp.where(mh, ph - (ph - fh), ph)
    pw2 = jnp.where(mw, pw - (pw - fw), pw)
    ph3 = jnp.clip(ph2, 0.0, hmax)
    pw3 = jnp.clip(pw2, 0.0, hmax)
    ghlt = 1.0 + (qlth - ph3)
    ghrb = 1.0 - (qrbh - ph3)
    gwlt = 1.0 + (qltw - pw3)
    gwrb = 1.0 - (qrbw - pw3)
    hb = jnp.minimum(qlth, hmax - 1.0)
    wb = jnp.minimum(qltw, hmax - 1.0)
    f32 = jnp.float32
    wh0 = ghlt * (qlth == hb).astype(f32) + ghrb * (qrbh == hb).astype(f32)
    wh1 = (ghlt * (qlth == hb + 1.0).astype(f32)
           + ghrb * (qrbh == hb + 1.0).astype(f32))
    ww0 = gwlt * (qltw == wb).astype(f32) + gwrb * (qrbw == wb).astype(f32)
    ww1 = (gwlt * (qltw == wb + 1.0).astype(f32)
           + gwrb * (qrbw == wb + 1.0).astype(f32))
    idx_ref[...] = (hb * jnp.float32(_HP) + wb).astype(jnp.int32)
    w00_ref[...] = wh0 * ww0
    w01_ref[...] = wh0 * ww1
    w10_ref[...] = wh1 * ww0
    w11_ref[...] = wh1 * ww1


def _idx_weights(off):
    spec_h = pl.BlockSpec((1, _H, _W), lambda k: (2 * k, 0, 0))
    spec_w = pl.BlockSpec((1, _H, _W), lambda k: (2 * k + 1, 0, 0))
    out_spec = pl.BlockSpec((1, _H, _W), lambda k: (k, 0, 0))
    sd = jax.ShapeDtypeStruct((_N, _H, _W), jnp.float32)
    return pl.pallas_call(
        _idx_weights_body,
        grid=(_N,),
        in_specs=[spec_h, spec_w],
        out_specs=[out_spec] * 5,
        out_shape=[jax.ShapeDtypeStruct((_N, _H, _W), jnp.int32),
                   sd, sd, sd, sd],
    )(off, off)


def _sc_gather(quads, idxf):
    mesh = plsc.VectorSubcoreMesh(core_axis_name="c", subcore_axis_name="s")
    grid = _NSAMP_PAD // _GWIN

    @functools.partial(
        pl.kernel,
        out_type=jax.ShapeDtypeStruct((_NSAMP_PAD, 4 * _C), jnp.float32),
        mesh=mesh,
    )
    def k(quads_hbm, idx_hbm, out_hbm):
        def body(i_vmem, o_vmem):
            pltpu.sync_copy(quads_hbm.at[i_vmem.at[0]], o_vmem)

        pltpu.emit_pipeline(
            body,
            grid=(grid,),
            in_specs=[pl.BlockSpec((1, _GWIN), lambda i: (0, i))],
            out_specs=[pl.BlockSpec((_GWIN, 4 * _C), lambda i: (i, 0))],
            core_axis_name=("c", "s"),
            dimension_semantics=(pltpu.PARALLEL,),
        )(idx_hbm, out_hbm)

    return k(quads, idxf)


def _combine_body(g_ref, wts_ref, w2_ref, out_ref):
    k = pl.program_id(1)

    @pl.when(k == 0)
    def _():
        out_ref[...] = jnp.zeros_like(out_ref)

    g = g_ref[...]
    wts = wts_ref[0]
    wk = w2_ref[0]
    acc = out_ref[...]
    for j in range(4):
        xj = g[:, j * _C:(j + 1) * _C] * wts[:, j:j + 1]
        acc = acc + jax.lax.dot_general(
            wk, xj, dimension_numbers=(((1,), (1,)), ((), ())),
            preferred_element_type=jnp.float32)
    out_ref[...] = acc


def _combine(gathered, wts, w2):
    return pl.pallas_call(
        _combine_body,
        grid=(_NT, _N),
        in_specs=[
            pl.BlockSpec((_T, 4 * _C), lambda t, k: (k * _NT + t, 0)),
            pl.BlockSpec((1, _T, 4), lambda t, k: (k, t, 0)),
            pl.BlockSpec((1, _C, _C), lambda t, k: (k, 0, 0)),
        ],
        out_specs=pl.BlockSpec((_C, _T), lambda t, k: (0, t)),
        out_shape=jax.ShapeDtypeStruct((_C, _NPIX), jnp.float32),
    )(gathered, wts, w2)


def kernel(x, offset, W):
    xp = jnp.pad(x[0], ((0, 0), (1, 1), (1, 1)))
    flat = xp.transpose(1, 2, 0).reshape(_HP * _HP, _C)
    flatp = jnp.pad(flat, ((0, _HP + 1), (0, 0)))
    npx = _HP * _HP
    quads = jnp.concatenate(
        [flatp[0:npx], flatp[1:npx + 1],
         flatp[_HP:npx + _HP], flatp[_HP + 1:npx + _HP + 1]], axis=1)

    off = offset[0].reshape(2 * _N, _H, _W)
    idx, w00, w01, w10, w11 = _idx_weights(off)

    idxf = jnp.concatenate(
        [idx.reshape(1, _NSAMP),
         jnp.zeros((1, _NSAMP_PAD - _NSAMP), jnp.int32)], axis=1)
    gathered = _sc_gather(quads, idxf)

    wts = jnp.stack([w00, w01, w10, w11], axis=-1).reshape(_N, _NPIX, 4)
    w2 = jnp.transpose(W.reshape(_C, _C, _N), (2, 0, 1))
    out = _combine(gathered, wts, w2)
    return out.reshape(1, _C, _H, _W)


# speedup vs baseline: 1416.6356x; 1416.6356x over previous
import functools

import jax
import jax.numpy as jnp
from jax.experimental import pallas as pl
from jax.experimental.pallas import tpu as pltpu
from jax.experimental.pallas import tpu_sc as plsc

_KS = 3
_N = _KS * _KS
_C = 96
_H = 224
_W = 224
_HP = _H + 2
_NPIX = _H * _W
_NSAMP = _N * _NPIX
_GWIN = 128
_NWORK = 32
_NSAMP_PAD = ((_NSAMP + _GWIN * _NWORK - 1) // (_GWIN * _NWORK)) * (_GWIN * _NWORK)
_T = 256
_NT = _NPIX // _T


def _idx_weights_body(offh_ref, offw_ref, idx_ref,
                      w00_ref, w01_ref, w10_ref, w11_ref):
    k = pl.program_id(0)
    dkx = (k // 3 - 1).astype(jnp.float32)
    dky = (k % 3 - 1).astype(jnp.float32)
    shape = offh_ref.shape
    hh = jax.lax.broadcasted_iota(jnp.int32, shape, 1).astype(jnp.float32)
    ww = jax.lax.broadcasted_iota(jnp.int32, shape, 2).astype(jnp.float32)
    hmax = jnp.float32(_HP - 1)
    ph = (hh + (1.0 + dkx)) + offh_ref[...]
    pw = (ww + (1.0 + dky)) + offw_ref[...]
    fh = jnp.floor(ph)
    fw = jnp.floor(pw)
    qlth = jnp.clip(fh, 0.0, hmax)
    qrbh = jnp.clip(fh + 1.0, 0.0, hmax)
    qltw = jnp.clip(fw, 0.0, hmax)
    qrbw = jnp.clip(fw + 1.0, 0.0, hmax)
    mh = jnp.logical_or(ph < 1.0, ph > hmax - 1.0)
    mw = jnp.logical_or(pw < 1.0, pw > hmax - 1.0)
    ph2 = jnp.where(mh, ph - (ph - fh), ph)
    pw2 = jnp.where(mw, pw - (pw - fw), pw)
    ph3 = jnp.clip(ph2, 0.0, hmax)
    pw3 = jnp.clip(pw2, 0.0, hmax)
    ghlt = 1.0 + (qlth - ph3)
    ghrb = 1.0 - (qrbh - ph3)
    gwlt = 1.0 + (qltw - pw3)
    gwrb = 1.0 - (qrbw - pw3)
    hb = jnp.minimum(qlth, hmax - 1.0)
    wb = jnp.minimum(qltw, hmax - 1.0)
    f32 = jnp.float32
    wh0 = ghlt * (qlth == hb).astype(f32) + ghrb * (qrbh == hb).astype(f32)
    wh1 = (ghlt * (qlth == hb + 1.0).astype(f32)
           + ghrb * (qrbh == hb + 1.0).astype(f32))
    ww0 = gwlt * (qltw == wb).astype(f32) + gwrb * (qrbw == wb).astype(f32)
    ww1 = (gwlt * (qltw == wb + 1.0).astype(f32)
           + gwrb * (qrbw == wb + 1.0).astype(f32))
    idx_ref[...] = (hb * jnp.float32(_HP) + wb).astype(jnp.int32)
    w00_ref[...] = wh0 * ww0
    w01_ref[...] = wh0 * ww1
    w10_ref[...] = wh1 * ww0
    w11_ref[...] = wh1 * ww1


def _idx_weights(off):
    spec_h = pl.BlockSpec((1, _H, _W), lambda k: (2 * k, 0, 0))
    spec_w = pl.BlockSpec((1, _H, _W), lambda k: (2 * k + 1, 0, 0))
    out_spec = pl.BlockSpec((1, _H, _W), lambda k: (k, 0, 0))
    sd = jax.ShapeDtypeStruct((_N, _H, _W), jnp.float32)
    return pl.pallas_call(
        _idx_weights_body,
        grid=(_N,),
        in_specs=[spec_h, spec_w],
        out_specs=[out_spec] * 5,
        out_shape=[jax.ShapeDtypeStruct((_N, _H, _W), jnp.int32),
                   sd, sd, sd, sd],
    )(off, off)


def _sc_gather(quads, idxf):
    mesh = plsc.VectorSubcoreMesh(core_axis_name="c", subcore_axis_name="s")
    grid = _NSAMP_PAD // _GWIN

    @functools.partial(
        pl.kernel,
        out_type=jax.ShapeDtypeStruct((_NSAMP_PAD, 4 * _C), jnp.float32),
        mesh=mesh,
    )
    def k(quads_hbm, idx_hbm, out_hbm):
        def body(i_vmem, o_vmem):
            pltpu.sync_copy(quads_hbm.at[i_vmem.at[0]], o_vmem)

        pltpu.emit_pipeline(
            body,
            grid=(grid,),
            in_specs=[pl.BlockSpec((1, _GWIN), lambda i: (0, i))],
            out_specs=[pl.BlockSpec((_GWIN, 4 * _C), lambda i: (i, 0))],
            core_axis_name=("c", "s"),
            dimension_semantics=(pltpu.PARALLEL,),
        )(idx_hbm, out_hbm)

    return k(quads, idxf)


def _combine_body(g_ref, wts_ref, w2_ref, out_ref):
    k = pl.program_id(1)

    @pl.when(k == 0)
    def _():
        out_ref[...] = jnp.zeros_like(out_ref)

    g = g_ref[...]
    wts = wts_ref[0]
    wk = w2_ref[0]
    acc = out_ref[...]
    for j in range(4):
        xj = g[:, j * _C:(j + 1) * _C] * wts[:, j:j + 1]
        acc = acc + jax.lax.dot_general(
            wk, xj, dimension_numbers=(((1,), (1,)), ((), ())),
            preferred_element_type=jnp.float32)
    out_ref[...] = acc


def _combine(gathered, wts, w2):
    return pl.pallas_call(
        _combine_body,
        grid=(_NT, _N),
        in_specs=[
            pl.BlockSpec((_T, 4 * _C), lambda t, k: (k * _NT + t, 0)),
            pl.BlockSpec((1, _T, 4), lambda t, k: (k, t, 0)),
            pl.BlockSpec((1, _C, _C), lambda t, k: (k, 0, 0)),
        ],
        out_specs=pl.BlockSpec((_C, _T), lambda t, k: (0, t)),
        out_shape=jax.ShapeDtypeStruct((_C, _NPIX), jnp.float32),
    )(gathered, wts, w2)


def kernel(x, offset, W):
    xp = jnp.pad(x[0], ((0, 0), (1, 1), (1, 1)))
    flat = xp.transpose(1, 2, 0).reshape(_HP * _HP, _C)
    flatp = jnp.pad(flat, ((0, _HP + 1), (0, 0)))
    npx = _HP * _HP
    quads = jnp.concatenate(
        [flatp[0:npx], flatp[1:npx + 1],
         flatp[_HP:npx + _HP], flatp[_HP + 1:npx + _HP + 1]], axis=1)

    off = offset[0].reshape(2 * _N, _H, _W)
    idx, w00, w01, w10, w11 = _idx_weights(off)

    idxf = jnp.concatenate(
        [idx.reshape(1, _NSAMP),
         jnp.zeros((1, _NSAMP_PAD - _NSAMP), jnp.int32)], axis=1)
    gathered = _sc_gather(quads, idxf)

    wts = jnp.stack([w00, w01, w10, w11], axis=-1).reshape(_N, _NPIX, 4)
    w2 = jnp.transpose(W.reshape(_C, _C, _N), (2, 0, 1))
    out = _combine(gathered, wts, w2)
    return out.reshape(1, _C, _H, _W)


# speedup vs baseline: 1793.3782x; 1.2659x over previous
import dataclasses
import functools

import jax
import jax.numpy as jnp
from jax.experimental import pallas as pl
from jax.experimental.pallas import tpu as pltpu
from jax.experimental.pallas import tpu_sc as plsc

_KS = 3
_N = _KS * _KS
_C = 96
_H = 224
_W = 224
_HP = _H + 2
_NPIX = _H * _W
_NSAMP = _N * _NPIX
_GWIN = 128
_NWORK = 32
_NSAMP_PAD = ((_NSAMP + _GWIN * _NWORK - 1) // (_GWIN * _NWORK)) * (_GWIN * _NWORK)
_T = 256
_NT = _NPIX // _T
_LANES = 16


def _idx_weights_body(offh_ref, offw_ref, idx_ref,
                      w00_ref, w01_ref, w10_ref, w11_ref):
    k = pl.program_id(0)
    dkx = (k // 3 - 1).astype(jnp.float32)
    dky = (k % 3 - 1).astype(jnp.float32)
    shape = offh_ref.shape
    hh = jax.lax.broadcasted_iota(jnp.int32, shape, 1).astype(jnp.float32)
    ww = jax.lax.broadcasted_iota(jnp.int32, shape, 2).astype(jnp.float32)
    hmax = jnp.float32(_HP - 1)
    ph = (hh + (1.0 + dkx)) + offh_ref[...]
    pw = (ww + (1.0 + dky)) + offw_ref[...]
    fh = jnp.floor(ph)
    fw = jnp.floor(pw)
    qlth = jnp.clip(fh, 0.0, hmax)
    qrbh = jnp.clip(fh + 1.0, 0.0, hmax)
    qltw = jnp.clip(fw, 0.0, hmax)
    qrbw = jnp.clip(fw + 1.0, 0.0, hmax)
    mh = jnp.logical_or(ph < 1.0, ph > hmax - 1.0)
    mw = jnp.logical_or(pw < 1.0, pw > hmax - 1.0)
    ph2 = jnp.where(mh, ph - (ph - fh), ph)
    pw2 = jnp.where(mw, pw - (pw - fw), pw)
    ph3 = jnp.clip(ph2, 0.0, hmax)
    pw3 = jnp.clip(pw2, 0.0, hmax)
    ghlt = 1.0 + (qlth - ph3)
    ghrb = 1.0 - (qrbh - ph3)
    gwlt = 1.0 + (qltw - pw3)
    gwrb = 1.0 - (qrbw - pw3)
    hb = jnp.minimum(qlth, hmax - 1.0)
    wb = jnp.minimum(qltw, hmax - 1.0)
    f32 = jnp.float32
    wh0 = ghlt * (qlth == hb).astype(f32) + ghrb * (qrbh == hb).astype(f32)
    wh1 = (ghlt * (qlth == hb + 1.0).astype(f32)
           + ghrb * (qrbh == hb + 1.0).astype(f32))
    ww0 = gwlt * (qltw == wb).astype(f32) + gwrb * (qrbw == wb).astype(f32)
    ww1 = (gwlt * (qltw == wb + 1.0).astype(f32)
           + gwrb * (qrbw == wb + 1.0).astype(f32))
    idx_ref[...] = (hb * jnp.float32(_HP) + wb).astype(jnp.int32)
    w00_ref[...] = wh0 * ww0
    w01_ref[...] = wh0 * ww1
    w10_ref[...] = wh1 * ww0
    w11_ref[...] = wh1 * ww1


def _idx_weights(off):
    spec_h = pl.BlockSpec((1, _H, _W), lambda k: (2 * k, 0, 0))
    spec_w = pl.BlockSpec((1, _H, _W), lambda k: (2 * k + 1, 0, 0))
    out_spec = pl.BlockSpec((1, _H, _W), lambda k: (k, 0, 0))
    sd = jax.ShapeDtypeStruct((_N, _H, _W), jnp.float32)
    return pl.pallas_call(
        _idx_weights_body,
        grid=(_N,),
        in_specs=[spec_h, spec_w],
        out_specs=[out_spec] * 5,
        out_shape=[jax.ShapeDtypeStruct((_N, _H, _W), jnp.int32),
                   sd, sd, sd, sd],
    )(off, off)


def _sc_gather_combine(quads, idxf, wtsf):
    mesh = plsc.VectorSubcoreMesh(core_axis_name="c", subcore_axis_name="s")
    grid = _NSAMP_PAD // _GWIN
    cp = pltpu.CompilerParams()
    if "needs_layout_passes" in pltpu.CompilerParams.__dataclass_fields__:
        cp = dataclasses.replace(cp, needs_layout_passes=False)

    @functools.partial(
        pl.kernel,
        out_type=jax.ShapeDtypeStruct((_NSAMP_PAD, _C), jnp.float32),
        mesh=mesh,
        scratch_types=[pltpu.VMEM((_GWIN, 4 * _C), jnp.float32)],
        compiler_params=cp,
    )
    def k(quads_hbm, idx_hbm, wts_hbm, out_hbm, gat):
        def body(i_vmem, w_vmem, o_vmem):
            pltpu.sync_copy(quads_hbm.at[i_vmem.at[0]], gat)

            @pl.loop(0, _GWIN)
            def _(i):
                isplat = jnp.full((_LANES,), i, jnp.int32)
                wv = [
                    plsc.load_gather(
                        w_vmem,
                        [jnp.full((_LANES,), j, jnp.int32), isplat])
                    for j in range(4)
                ]
                for v in range(_C // _LANES):
                    acc = wv[0] * gat[i, pl.ds(v * _LANES, _LANES)]
                    for j in range(1, 4):
                        acc = acc + wv[j] * gat[
                            i, pl.ds(j * _C + v * _LANES, _LANES)]
                    o_vmem[i, pl.ds(v * _LANES, _LANES)] = acc

        pltpu.emit_pipeline(
            body,
            grid=(grid,),
            in_specs=[pl.BlockSpec((1, _GWIN), lambda i: (0, i)),
                      pl.BlockSpec((4, _GWIN), lambda i: (0, i))],
            out_specs=[pl.BlockSpec((_GWIN, _C), lambda i: (i, 0))],
            core_axis_name=("c", "s"),
            dimension_semantics=(pltpu.PARALLEL,),
        )(idx_hbm, wts_hbm, out_hbm)

    return k(quads, idxf, wtsf)


def _combine_body(g_ref, w2_ref, out_ref):
    k = pl.program_id(1)

    @pl.when(k == 0)
    def _():
        out_ref[...] = jnp.zeros_like(out_ref)

    g = g_ref[...]
    wk = w2_ref[0]
    out_ref[...] += jax.lax.dot_general(
        wk, g, dimension_numbers=(((1,), (1,)), ((), ())),
        preferred_element_type=jnp.float32)


def _combine(xoff, w2):
    return pl.pallas_call(
        _combine_body,
        grid=(_NT, _N),
        in_specs=[
            pl.BlockSpec((_T, _C), lambda t, k: (k * _NT + t, 0)),
            pl.BlockSpec((1, _C, _C), lambda t, k: (k, 0, 0)),
        ],
        out_specs=pl.BlockSpec((_C, _T), lambda t, k: (0, t)),
        out_shape=jax.ShapeDtypeStruct((_C, _NPIX), jnp.float32),
    )(xoff, w2)


def kernel(x, offset, W):
    xp = jnp.pad(x[0], ((0, 0), (1, 1), (1, 1)))
    flat = xp.transpose(1, 2, 0).reshape(_HP * _HP, _C)
    flatp = jnp.pad(flat, ((0, _HP + 1), (0, 0)))
    npx = _HP * _HP
    quads = jnp.concatenate(
        [flatp[0:npx], flatp[1:npx + 1],
         flatp[_HP:npx + _HP], flatp[_HP + 1:npx + _HP + 1]], axis=1)

    off = offset[0].reshape(2 * _N, _H, _W)
    idx, w00, w01, w10, w11 = _idx_weights(off)

    pad = _NSAMP_PAD - _NSAMP
    idxf = jnp.concatenate(
        [idx.reshape(1, _NSAMP), jnp.zeros((1, pad), jnp.int32)], axis=1)
    wtsf = jnp.concatenate(
        [jnp.stack([w00, w01, w10, w11]).reshape(4, _NSAMP),
         jnp.zeros((4, pad), jnp.float32)], axis=1)
    xoff = _sc_gather_combine(quads, idxf, wtsf)

    w2 = jnp.transpose(W.reshape(_C, _C, _N), (2, 0, 1))
    out = _combine(xoff, w2)
    return out.reshape(1, _C, _H, _W)


# speedup vs baseline: 2848.8186x; 1.5885x over previous
import dataclasses
import functools

import jax
import jax.numpy as jnp
from jax.experimental import pallas as pl
from jax.experimental.pallas import tpu as pltpu
from jax.experimental.pallas import tpu_sc as plsc

_KS = 3
_N = _KS * _KS
_C = 96
_H = 224
_W = 224
_HP = _H + 2
_NPIX = _H * _W
_NSAMP = _N * _NPIX
_GWIN = 128
_NWORK = 32
_NSAMP_PAD = ((_NSAMP + _GWIN * _NWORK - 1) // (_GWIN * _NWORK)) * (_GWIN * _NWORK)
_T = 3584
_NT = _NPIX // _T
_LANES = 16


def _idx_weights_body(offh_ref, offw_ref, idx_ref,
                      w00_ref, w01_ref, w10_ref, w11_ref):
    k = pl.program_id(0)
    dkx = (k // 3 - 1).astype(jnp.float32)
    dky = (k % 3 - 1).astype(jnp.float32)
    shape = offh_ref.shape
    hh = jax.lax.broadcasted_iota(jnp.int32, shape, 1).astype(jnp.float32)
    ww = jax.lax.broadcasted_iota(jnp.int32, shape, 2).astype(jnp.float32)
    hmax = jnp.float32(_HP - 1)
    ph = (hh + (1.0 + dkx)) + offh_ref[...]
    pw = (ww + (1.0 + dky)) + offw_ref[...]
    fh = jnp.floor(ph)
    fw = jnp.floor(pw)
    qlth = jnp.clip(fh, 0.0, hmax)
    qrbh = jnp.clip(fh + 1.0, 0.0, hmax)
    qltw = jnp.clip(fw, 0.0, hmax)
    qrbw = jnp.clip(fw + 1.0, 0.0, hmax)
    mh = jnp.logical_or(ph < 1.0, ph > hmax - 1.0)
    mw = jnp.logical_or(pw < 1.0, pw > hmax - 1.0)
    ph2 = jnp.where(mh, ph - (ph - fh), ph)
    pw2 = jnp.where(mw, pw - (pw - fw), pw)
    ph3 = jnp.clip(ph2, 0.0, hmax)
    pw3 = jnp.clip(pw2, 0.0, hmax)
    ghlt = 1.0 + (qlth - ph3)
    ghrb = 1.0 - (qrbh - ph3)
    gwlt = 1.0 + (qltw - pw3)
    gwrb = 1.0 - (qrbw - pw3)
    hb = jnp.minimum(qlth, hmax - 1.0)
    wb = jnp.minimum(qltw, hmax - 1.0)
    f32 = jnp.float32
    wh0 = ghlt * (qlth == hb).astype(f32) + ghrb * (qrbh == hb).astype(f32)
    wh1 = (ghlt * (qlth == hb + 1.0).astype(f32)
           + ghrb * (qrbh == hb + 1.0).astype(f32))
    ww0 = gwlt * (qltw == wb).astype(f32) + gwrb * (qrbw == wb).astype(f32)
    ww1 = (gwlt * (qltw == wb + 1.0).astype(f32)
           + gwrb * (qrbw == wb + 1.0).astype(f32))
    idx_ref[...] = (hb * jnp.float32(_HP) + wb).astype(jnp.int32)
    w00_ref[...] = wh0 * ww0
    w01_ref[...] = wh0 * ww1
    w10_ref[...] = wh1 * ww0
    w11_ref[...] = wh1 * ww1


def _idx_weights(off):
    spec_h = pl.BlockSpec((1, _H, _W), lambda k: (2 * k, 0, 0))
    spec_w = pl.BlockSpec((1, _H, _W), lambda k: (2 * k + 1, 0, 0))
    out_spec = pl.BlockSpec((1, _H, _W), lambda k: (k, 0, 0))
    sd = jax.ShapeDtypeStruct((_N, _H, _W), jnp.float32)
    return pl.pallas_call(
        _idx_weights_body,
        grid=(_N,),
        in_specs=[spec_h, spec_w],
        out_specs=[out_spec] * 5,
        out_shape=[jax.ShapeDtypeStruct((_N, _H, _W), jnp.int32),
                   sd, sd, sd, sd],
    )(off, off)


def _sc_gather_combine(quads, idxf, wtsf):
    mesh = plsc.VectorSubcoreMesh(core_axis_name="c", subcore_axis_name="s")
    grid = _NSAMP_PAD // _GWIN
    cp = pltpu.CompilerParams()
    if "needs_layout_passes" in pltpu.CompilerParams.__dataclass_fields__:
        cp = dataclasses.replace(cp, needs_layout_passes=False)

    @functools.partial(
        pl.kernel,
        out_type=jax.ShapeDtypeStruct((_NSAMP_PAD, _C), jnp.float32),
        mesh=mesh,
        scratch_types=[pltpu.VMEM((_GWIN, 4 * _C), jnp.float32)],
        compiler_params=cp,
    )
    def k(quads_hbm, idx_hbm, wts_hbm, out_hbm, gat):
        def body(i_vmem, w_vmem, o_vmem):
            pltpu.sync_copy(quads_hbm.at[i_vmem.at[0]], gat)

            @pl.loop(0, _GWIN)
            def _(i):
                isplat = jnp.full((_LANES,), i, jnp.int32)
                wv = [
                    plsc.load_gather(
                        w_vmem,
                        [jnp.full((_LANES,), j, jnp.int32), isplat])
                    for j in range(4)
                ]
                for v in range(_C // _LANES):
                    acc = wv[0] * gat[i, pl.ds(v * _LANES, _LANES)]
                    for j in range(1, 4):
                        acc = acc + wv[j] * gat[
                            i, pl.ds(j * _C + v * _LANES, _LANES)]
                    o_vmem[i, pl.ds(v * _LANES, _LANES)] = acc

        pltpu.emit_pipeline(
            body,
            grid=(grid,),
            in_specs=[pl.BlockSpec((1, _GWIN), lambda i: (0, i)),
                      pl.BlockSpec((4, _GWIN), lambda i: (0, i))],
            out_specs=[pl.BlockSpec((_GWIN, _C), lambda i: (i, 0))],
            core_axis_name=("c", "s"),
            dimension_semantics=(pltpu.PARALLEL,),
        )(idx_hbm, wts_hbm, out_hbm)

    return k(quads, idxf, wtsf)


def _combine_body(*refs):
    gs = refs[:_N]
    w2_ref = refs[_N]
    out_ref = refs[_N + 1]
    acc = jax.lax.dot_general(
        w2_ref[0], gs[0][...], dimension_numbers=(((1,), (1,)), ((), ())),
        preferred_element_type=jnp.float32)
    for k in range(1, _N):
        acc = acc + jax.lax.dot_general(
            w2_ref[k], gs[k][...], dimension_numbers=(((1,), (1,)), ((), ())),
            preferred_element_type=jnp.float32)
    out_ref[...] = acc


def _combine(xoff, w2):
    in_specs = [
        pl.BlockSpec((_T, _C), functools.partial(
            lambda k, t: (k * _NT + t, 0), k))
        for k in range(_N)
    ]
    in_specs.append(pl.BlockSpec((_N, _C, _C), lambda t: (0, 0, 0)))
    return pl.pallas_call(
        _combine_body,
        grid=(_NT,),
        in_specs=in_specs,
        out_specs=pl.BlockSpec((_C, _T), lambda t: (0, t)),
        out_shape=jax.ShapeDtypeStruct((_C, _NPIX), jnp.float32),
    )(*([xoff] * _N), w2)


def kernel(x, offset, W):
    xp = jnp.pad(x[0], ((0, 0), (1, 1), (1, 1)))
    flat = xp.transpose(1, 2, 0).reshape(_HP * _HP, _C)
    flatp = jnp.pad(flat, ((0, _HP + 1), (0, 0)))
    npx = _HP * _HP
    quads = jnp.concatenate(
        [flatp[0:npx], flatp[1:npx + 1],
         flatp[_HP:npx + _HP], flatp[_HP + 1:npx + _HP + 1]], axis=1)

    off = offset[0].reshape(2 * _N, _H, _W)
    idx, w00, w01, w10, w11 = _idx_weights(off)

    pad = _NSAMP_PAD - _NSAMP
    idxf = jnp.concatenate(
        [idx.reshape(1, _NSAMP), jnp.zeros((1, pad), jnp.int32)], axis=1)
    wtsf = jnp.concatenate(
        [jnp.stack([w00, w01, w10, w11]).reshape(4, _NSAMP),
         jnp.zeros((4, pad), jnp.float32)], axis=1)
    xoff = _sc_gather_combine(quads, idxf, wtsf)

    w2 = jnp.transpose(W.reshape(_C, _C, _N), (2, 0, 1))
    out = _combine(xoff, w2)
    return out.reshape(1, _C, _H, _W)


# speedup vs baseline: 3478.5640x; 1.2211x over previous
import dataclasses
import functools

import jax
import jax.numpy as jnp
from jax.experimental import pallas as pl
from jax.experimental.pallas import tpu as pltpu
from jax.experimental.pallas import tpu_sc as plsc

_KS = 3
_N = _KS * _KS
_C = 96
_H = 224
_W = 224
_HP = _H + 2
_NPIX = _H * _W
_NSAMP = _N * _NPIX
_GWIN = 128
_NWORK = 32
_NSAMP_PAD = (
    (_NSAMP + 2 * _GWIN * _NWORK - 1)
    // (2 * _GWIN * _NWORK) * (2 * _GWIN * _NWORK))
_WPW = _NSAMP_PAD // (_GWIN * _NWORK)
_T = 3584
_NT = _NPIX // _T
_LANES = 16


def _idx_weights_body(offh_ref, offw_ref, idx_ref,
                      w00_ref, w01_ref, w10_ref, w11_ref):
    k = pl.program_id(0)
    dkx = (k // 3 - 1).astype(jnp.float32)
    dky = (k % 3 - 1).astype(jnp.float32)
    shape = offh_ref.shape
    hh = jax.lax.broadcasted_iota(jnp.int32, shape, 1).astype(jnp.float32)
    ww = jax.lax.broadcasted_iota(jnp.int32, shape, 2).astype(jnp.float32)
    hmax = jnp.float32(_HP - 1)
    ph = (hh + (1.0 + dkx)) + offh_ref[...]
    pw = (ww + (1.0 + dky)) + offw_ref[...]
    fh = jnp.floor(ph)
    fw = jnp.floor(pw)
    qlth = jnp.clip(fh, 0.0, hmax)
    qrbh = jnp.clip(fh + 1.0, 0.0, hmax)
    qltw = jnp.clip(fw, 0.0, hmax)
    qrbw = jnp.clip(fw + 1.0, 0.0, hmax)
    mh = jnp.logical_or(ph < 1.0, ph > hmax - 1.0)
    mw = jnp.logical_or(pw < 1.0, pw > hmax - 1.0)
    ph2 = jnp.where(mh, ph - (ph - fh), ph)
    pw2 = jnp.where(mw, pw - (pw - fw), pw)
    ph3 = jnp.clip(ph2, 0.0, hmax)
    pw3 = jnp.clip(pw2, 0.0, hmax)
    ghlt = 1.0 + (qlth - ph3)
    ghrb = 1.0 - (qrbh - ph3)
    gwlt = 1.0 + (qltw - pw3)
    gwrb = 1.0 - (qrbw - pw3)
    hb = jnp.minimum(qlth, hmax - 1.0)
    wb = jnp.minimum(qltw, hmax - 1.0)
    f32 = jnp.float32
    wh0 = ghlt * (qlth == hb).astype(f32) + ghrb * (qrbh == hb).astype(f32)
    wh1 = (ghlt * (qlth == hb + 1.0).astype(f32)
           + ghrb * (qrbh == hb + 1.0).astype(f32))
    ww0 = gwlt * (qltw == wb).astype(f32) + gwrb * (qrbw == wb).astype(f32)
    ww1 = (gwlt * (qltw == wb + 1.0).astype(f32)
           + gwrb * (qrbw == wb + 1.0).astype(f32))
    idx_ref[...] = (hb * jnp.float32(_HP) + wb).astype(jnp.int32)
    w00_ref[...] = wh0 * ww0
    w01_ref[...] = wh0 * ww1
    w10_ref[...] = wh1 * ww0
    w11_ref[...] = wh1 * ww1


def _idx_weights(off):
    spec_h = pl.BlockSpec((1, _H, _W), lambda k: (2 * k, 0, 0))
    spec_w = pl.BlockSpec((1, _H, _W), lambda k: (2 * k + 1, 0, 0))
    out_spec = pl.BlockSpec((1, _H, _W), lambda k: (k, 0, 0))
    sd = jax.ShapeDtypeStruct((_N, _H, _W), jnp.float32)
    return pl.pallas_call(
        _idx_weights_body,
        grid=(_N,),
        in_specs=[spec_h, spec_w],
        out_specs=[out_spec] * 5,
        out_shape=[jax.ShapeDtypeStruct((_N, _H, _W), jnp.int32),
                   sd, sd, sd, sd],
    )(off, off)


def _sc_gather_combine(quads, idxf, wtsf):
    mesh = plsc.VectorSubcoreMesh(core_axis_name="c", subcore_axis_name="s")
    cp = pltpu.CompilerParams()
    if "needs_layout_passes" in pltpu.CompilerParams.__dataclass_fields__:
        cp = dataclasses.replace(cp, needs_layout_passes=False)

    @functools.partial(
        pl.kernel,
        out_type=jax.ShapeDtypeStruct((_NSAMP_PAD, _C), jnp.float32),
        mesh=mesh,
        scratch_types=[
            pltpu.VMEM((2, _GWIN), jnp.int32),
            pltpu.VMEM((2, 4, _GWIN), jnp.float32),
            pltpu.VMEM((2, _GWIN, 4 * _C), jnp.float32),
            pltpu.VMEM((_GWIN, _C), jnp.float32),
            pltpu.SemaphoreType.DMA((2,)),
            pltpu.SemaphoreType.DMA((2,)),
            pltpu.SemaphoreType.DMA((2,)),
            pltpu.SemaphoreType.DMA,
        ],
        compiler_params=cp,
    )
    def k(quads_hbm, idx_hbm, wts_hbm, out_hbm,
          idxb, wtsb, gatb, outb, sidx, swts, sgat, sout):
        wid = (jax.lax.axis_index("s") * 2
               + jax.lax.axis_index("c")).astype(jnp.int32)
        base = wid * _WPW

        def idx_copy(w, b):
            return pltpu.make_async_copy(
                idx_hbm.at[w], idxb.at[b], sidx.at[b])

        def wts_copy(w, b):
            return pltpu.make_async_copy(
                wts_hbm.at[w], wtsb.at[b], swts.at[b])

        def gat_copy(b):
            return pltpu.make_async_copy(
                quads_hbm.at[idxb.at[b]], gatb.at[b], sgat.at[b])

        def out_copy(w):
            return pltpu.make_async_copy(
                outb, out_hbm.at[pl.ds(w * _GWIN, _GWIN)], sout)

        for b in range(2):
            idx_copy(base + b, b).start()
            wts_copy(base + b, b).start()
        for b in range(2):
            idx_copy(base + b, b).wait()
            wts_copy(base + b, b).wait()
            gat_copy(b).start()

        last = _WPW // 2 - 1

        @pl.loop(0, _WPW // 2)
        def _(it):
            for b in range(2):
                w = base + it * 2 + b
                gat_copy(b).wait()

                @pl.when(it < last)
                def _():
                    idx_copy(w + 2, b).start()

                @pl.when(w > base)
                def _():
                    out_copy(w - 1).wait()

                @pl.when(it > 0)
                def _():
                    wts_copy(w, b).wait()

                @pl.loop(0, _GWIN)
                def _(i):
                    isplat = jnp.full((_LANES,), i, jnp.int32)
                    wv = [
                        plsc.load_gather(
                            wtsb.at[b],
                            [jnp.full((_LANES,), j, jnp.int32), isplat])
                        for j in range(4)
                    ]
                    for v in range(_C // _LANES):
                        acc = wv[0] * gatb[b, i, pl.ds(v * _LANES, _LANES)]
                        for j in range(1, 4):
                            acc = acc + wv[j] * gatb[
                                b, i, pl.ds(j * _C + v * _LANES, _LANES)]
                        outb[i, pl.ds(v * _LANES, _LANES)] = acc

                @pl.when(it < last)
                def _():
                    wts_copy(w + 2, b).start()

                out_copy(w).start()

                @pl.when(it < last)
                def _():
                    idx_copy(w + 2, b).wait()
                    gat_copy(b).start()

        out_copy(base + _WPW - 1).wait()

    return k(quads, idxf, wtsf)


def _combine_body(*refs):
    gs = refs[:_N]
    w2_ref = refs[_N]
    out_ref = refs[_N + 1]
    acc = jax.lax.dot_general(
        w2_ref[0], gs[0][...], dimension_numbers=(((1,), (1,)), ((), ())),
        preferred_element_type=jnp.float32)
    for k in range(1, _N):
        acc = acc + jax.lax.dot_general(
            w2_ref[k], gs[k][...], dimension_numbers=(((1,), (1,)), ((), ())),
            preferred_element_type=jnp.float32)
    out_ref[...] = acc


def _combine(xoff, w2):
    in_specs = [
        pl.BlockSpec((_T, _C), functools.partial(
            lambda k, t: (k * _NT + t, 0), k))
        for k in range(_N)
    ]
    in_specs.append(pl.BlockSpec((_N, _C, _C), lambda t: (0, 0, 0)))
    return pl.pallas_call(
        _combine_body,
        grid=(_NT,),
        in_specs=in_specs,
        out_specs=pl.BlockSpec((_C, _T), lambda t: (0, t)),
        out_shape=jax.ShapeDtypeStruct((_C, _NPIX), jnp.float32),
    )(*([xoff] * _N), w2)


def kernel(x, offset, W):
    xp = jnp.pad(x[0], ((0, 0), (1, 1), (1, 1)))
    flat = xp.transpose(1, 2, 0).reshape(_HP * _HP, _C)
    flatp = jnp.pad(flat, ((0, _HP + 1), (0, 0)))
    npx = _HP * _HP
    quads = jnp.concatenate(
        [flatp[0:npx], flatp[1:npx + 1],
         flatp[_HP:npx + _HP], flatp[_HP + 1:npx + _HP + 1]], axis=1)

    off = offset[0].reshape(2 * _N, _H, _W)
    idx, w00, w01, w10, w11 = _idx_weights(off)

    nwin = _NSAMP_PAD // _GWIN
    pad = _NSAMP_PAD - _NSAMP
    idxf = jnp.concatenate(
        [idx.reshape(1, _NSAMP), jnp.zeros((1, pad), jnp.int32)],
        axis=1).reshape(nwin, _GWIN)
    wtsf = jnp.concatenate(
        [jnp.stack([w00, w01, w10, w11]).reshape(4, _NSAMP),
         jnp.zeros((4, pad), jnp.float32)],
        axis=1).reshape(4, nwin, _GWIN).transpose(1, 0, 2)
    xoff = _sc_gather_combine(quads, idxf, wtsf)

    w2 = jnp.transpose(W.reshape(_C, _C, _N), (2, 0, 1))
    out = _combine(xoff, w2)
    return out.reshape(1, _C, _H, _W)


# speedup vs baseline: 3491.1973x; 1.0036x over previous
import dataclasses
import functools

import jax
import jax.numpy as jnp
from jax.experimental import pallas as pl
from jax.experimental.pallas import tpu as pltpu
from jax.experimental.pallas import tpu_sc as plsc

_KS = 3
_N = _KS * _KS
_C = 96
_H = 224
_W = 224
_HP = _H + 2
_NPIX = _H * _W
_NSAMP = _N * _NPIX
_GWIN = 128
_NWORK = 32
_NSAMP_PAD = (
    (_NSAMP + 2 * _GWIN * _NWORK - 1)
    // (2 * _GWIN * _NWORK) * (2 * _GWIN * _NWORK))
_WPW = _NSAMP_PAD // (_GWIN * _NWORK)
_T = 3584
_NT = _NPIX // _T
_LANES = 16


def _idx_weights_body(offh_ref, offw_ref, idx_ref, wts_ref):
    k = pl.program_id(0)
    dkx = (k // 3 - 1).astype(jnp.float32)
    dky = (k % 3 - 1).astype(jnp.float32)
    shape = offh_ref.shape
    pos = (jax.lax.broadcasted_iota(jnp.int32, shape, 1) * _GWIN
           + jax.lax.broadcasted_iota(jnp.int32, shape, 2))
    hi = jax.lax.shift_right_logical(
        jax.lax.shift_right_logical(pos, 5) * 9363, 16)
    hh = hi.astype(jnp.float32)
    ww = (pos - hi * 224).astype(jnp.float32)
    hmax = jnp.float32(_HP - 1)
    ph = (hh + (1.0 + dkx)) + offh_ref[...]
    pw = (ww + (1.0 + dky)) + offw_ref[...]
    fh = jnp.floor(ph)
    fw = jnp.floor(pw)
    qlth = jnp.clip(fh, 0.0, hmax)
    qrbh = jnp.clip(fh + 1.0, 0.0, hmax)
    qltw = jnp.clip(fw, 0.0, hmax)
    qrbw = jnp.clip(fw + 1.0, 0.0, hmax)
    mh = jnp.logical_or(ph < 1.0, ph > hmax - 1.0)
    mw = jnp.logical_or(pw < 1.0, pw > hmax - 1.0)
    ph2 = jnp.where(mh, ph - (ph - fh), ph)
    pw2 = jnp.where(mw, pw - (pw - fw), pw)
    ph3 = jnp.clip(ph2, 0.0, hmax)
    pw3 = jnp.clip(pw2, 0.0, hmax)
    ghlt = 1.0 + (qlth - ph3)
    ghrb = 1.0 - (qrbh - ph3)
    gwlt = 1.0 + (qltw - pw3)
    gwrb = 1.0 - (qrbw - pw3)
    hb = jnp.minimum(qlth, hmax - 1.0)
    wb = jnp.minimum(qltw, hmax - 1.0)
    f32 = jnp.float32
    wh0 = ghlt * (qlth == hb).astype(f32) + ghrb * (qrbh == hb).astype(f32)
    wh1 = (ghlt * (qlth == hb + 1.0).astype(f32)
           + ghrb * (qrbh == hb + 1.0).astype(f32))
    ww0 = gwlt * (qltw == wb).astype(f32) + gwrb * (qrbw == wb).astype(f32)
    ww1 = (gwlt * (qltw == wb + 1.0).astype(f32)
           + gwrb * (qrbw == wb + 1.0).astype(f32))
    idx_ref[...] = (hb * jnp.float32(_HP) + wb).astype(jnp.int32)
    for j, wj in enumerate([wh0 * ww0, wh0 * ww1, wh1 * ww0, wh1 * ww1]):
        wts_ref[:, :, j, :] = wj


def _idx_weights(off):
    nw = _NPIX // _GWIN
    spec_h = pl.BlockSpec((1, nw, _GWIN), lambda k: (2 * k, 0, 0))
    spec_w = pl.BlockSpec((1, nw, _GWIN), lambda k: (2 * k + 1, 0, 0))
    return pl.pallas_call(
        _idx_weights_body,
        grid=(_N,),
        in_specs=[spec_h, spec_w],
        out_specs=[pl.BlockSpec((1, nw, _GWIN), lambda k: (k, 0, 0)),
                   pl.BlockSpec((1, nw, 4, _GWIN), lambda k: (k, 0, 0, 0))],
        out_shape=[jax.ShapeDtypeStruct((_N, nw, _GWIN), jnp.int32),
                   jax.ShapeDtypeStruct((_N, nw, 4, _GWIN), jnp.float32)],
    )(off, off)


def _sc_gather_combine(quads, idxf, wtsf):
    mesh = plsc.VectorSubcoreMesh(core_axis_name="c", subcore_axis_name="s")
    cp = pltpu.CompilerParams()
    if "needs_layout_passes" in pltpu.CompilerParams.__dataclass_fields__:
        cp = dataclasses.replace(cp, needs_layout_passes=False)

    @functools.partial(
        pl.kernel,
        out_type=jax.ShapeDtypeStruct((_NSAMP_PAD, _C), jnp.float32),
        mesh=mesh,
        scratch_types=[
            pltpu.VMEM((2, _GWIN), jnp.int32),
            pltpu.VMEM((2, 4, _GWIN), jnp.float32),
            pltpu.VMEM((2, _GWIN, 4 * _C), jnp.float32),
            pltpu.VMEM((_GWIN, _C), jnp.float32),
            pltpu.SemaphoreType.DMA((2,)),
            pltpu.SemaphoreType.DMA((2,)),
            pltpu.SemaphoreType.DMA((2,)),
            pltpu.SemaphoreType.DMA,
        ],
        compiler_params=cp,
    )
    def k(quads_hbm, idx_hbm, wts_hbm, out_hbm,
          idxb, wtsb, gatb, outb, sidx, swts, sgat, sout):
        wid = (jax.lax.axis_index("s") * 2
               + jax.lax.axis_index("c")).astype(jnp.int32)
        base = wid * _WPW

        def idx_copy(w, b):
            return pltpu.make_async_copy(
                idx_hbm.at[w], idxb.at[b], sidx.at[b])

        def wts_copy(w, b):
            return pltpu.make_async_copy(
                wts_hbm.at[w], wtsb.at[b], swts.at[b])

        def gat_copy(b):
            return pltpu.make_async_copy(
                quads_hbm.at[idxb.at[b]], gatb.at[b], sgat.at[b])

        def out_copy(w):
            return pltpu.make_async_copy(
                outb, out_hbm.at[pl.ds(w * _GWIN, _GWIN)], sout)

        for b in range(2):
            idx_copy(base + b, b).start()
            wts_copy(base + b, b).start()
        for b in range(2):
            idx_copy(base + b, b).wait()
            wts_copy(base + b, b).wait()
            gat_copy(b).start()

        last = _WPW // 2 - 1

        @pl.loop(0, _WPW // 2)
        def _(it):
            for b in range(2):
                w = base + it * 2 + b
                gat_copy(b).wait()

                @pl.when(it < last)
                def _():
                    idx_copy(w + 2, b).start()

                @pl.when(w > base)
                def _():
                    out_copy(w - 1).wait()

                @pl.when(it > 0)
                def _():
                    wts_copy(w, b).wait()

                @pl.loop(0, _GWIN)
                def _(i):
                    isplat = jnp.full((_LANES,), i, jnp.int32)
                    wv = [
                        plsc.load_gather(
                            wtsb.at[b],
                            [jnp.full((_LANES,), j, jnp.int32), isplat])
                        for j in range(4)
                    ]
                    for v in range(_C // _LANES):
                        acc = wv[0] * gatb[b, i, pl.ds(v * _LANES, _LANES)]
                        for j in range(1, 4):
                            acc = acc + wv[j] * gatb[
                                b, i, pl.ds(j * _C + v * _LANES, _LANES)]
                        outb[i, pl.ds(v * _LANES, _LANES)] = acc

                @pl.when(it < last)
                def _():
                    wts_copy(w + 2, b).start()

                out_copy(w).start()

                @pl.when(it < last)
                def _():
                    idx_copy(w + 2, b).wait()
                    gat_copy(b).start()

        out_copy(base + _WPW - 1).wait()

    return k(quads, idxf, wtsf)


def _combine_body(*refs):
    gs = refs[:_N]
    w2_ref = refs[_N]
    out_ref = refs[_N + 1]
    acc = jax.lax.dot_general(
        w2_ref[0], gs[0][...], dimension_numbers=(((1,), (1,)), ((), ())),
        preferred_element_type=jnp.float32)
    for k in range(1, _N):
        acc = acc + jax.lax.dot_general(
            w2_ref[k], gs[k][...], dimension_numbers=(((1,), (1,)), ((), ())),
            preferred_element_type=jnp.float32)
    out_ref[...] = acc


def _combine(xoff, w2):
    in_specs = [
        pl.BlockSpec((_T, _C), functools.partial(
            lambda k, t: (k * _NT + t, 0), k))
        for k in range(_N)
    ]
    in_specs.append(pl.BlockSpec((_N, _C, _C), lambda t: (0, 0, 0)))
    return pl.pallas_call(
        _combine_body,
        grid=(_NT,),
        in_specs=in_specs,
        out_specs=pl.BlockSpec((_C, _T), lambda t: (0, t)),
        out_shape=jax.ShapeDtypeStruct((_C, _NPIX), jnp.float32),
    )(*([xoff] * _N), w2)


def kernel(x, offset, W):
    xp = jnp.pad(x[0], ((0, 0), (1, 1), (1, 1)))
    flat = xp.transpose(1, 2, 0).reshape(_HP * _HP, _C)
    flatp = jnp.pad(flat, ((0, _HP + 1), (0, 0)))
    npx = _HP * _HP
    quads = jnp.concatenate(
        [flatp[0:npx], flatp[1:npx + 1],
         flatp[_HP:npx + _HP], flatp[_HP + 1:npx + _HP + 1]], axis=1)

    off = offset[0].reshape(2 * _N, _NPIX // _GWIN, _GWIN)
    idx, wts = _idx_weights(off)

    nwin = _NSAMP_PAD // _GWIN
    npadw = nwin - _N * (_NPIX // _GWIN)
    idxf = jnp.concatenate(
        [idx.reshape(-1, _GWIN), jnp.zeros((npadw, _GWIN), jnp.int32)])
    wtsf = jnp.concatenate(
        [wts.reshape(-1, 4, _GWIN),
         jnp.zeros((npadw, 4, _GWIN), jnp.float32)])
    xoff = _sc_gather_combine(quads, idxf, wtsf)

    w2 = jnp.transpose(W.reshape(_C, _C, _N), (2, 0, 1))
    out = _combine(xoff, w2)
    return out.reshape(1, _C, _H, _W)


# speedup vs baseline: 3680.2573x; 1.0542x over previous
import dataclasses
import functools

import jax
import jax.numpy as jnp
from jax.experimental import pallas as pl
from jax.experimental.pallas import tpu as pltpu
from jax.experimental.pallas import tpu_sc as plsc

_KS = 3
_N = _KS * _KS
_C = 96
_H = 224
_W = 224
_HP = _H + 2
_NPIX = _H * _W
_NSAMP = _N * _NPIX
_GWIN = 128
_NWORK = 32
_NSAMP_PAD = (
    (_NSAMP + 2 * _GWIN * _NWORK - 1)
    // (2 * _GWIN * _NWORK) * (2 * _GWIN * _NWORK))
_WPW = _NSAMP_PAD // (_GWIN * _NWORK)
_T = 3584
_NT = _NPIX // _T
_LANES = 16


def _idx_weights_body(offh_ref, offw_ref, idx_ref, wts_ref):
    k = pl.program_id(0)
    dkx = (k // 3 - 1).astype(jnp.float32)
    dky = (k % 3 - 1).astype(jnp.float32)
    shape = offh_ref.shape
    pos = (jax.lax.broadcasted_iota(jnp.int32, shape, 1) * _GWIN
           + jax.lax.broadcasted_iota(jnp.int32, shape, 2))
    hi = jax.lax.shift_right_logical(
        jax.lax.shift_right_logical(pos, 5) * 9363, 16)
    hh = hi.astype(jnp.float32)
    ww = (pos - hi * 224).astype(jnp.float32)
    hmax = jnp.float32(_HP - 1)
    ph = (hh + (1.0 + dkx)) + offh_ref[...]
    pw = (ww + (1.0 + dky)) + offw_ref[...]
    fh = jnp.floor(ph)
    fw = jnp.floor(pw)
    qlth = jnp.clip(fh, 0.0, hmax)
    qrbh = jnp.clip(fh + 1.0, 0.0, hmax)
    qltw = jnp.clip(fw, 0.0, hmax)
    qrbw = jnp.clip(fw + 1.0, 0.0, hmax)
    mh = jnp.logical_or(ph < 1.0, ph > hmax - 1.0)
    mw = jnp.logical_or(pw < 1.0, pw > hmax - 1.0)
    ph2 = jnp.where(mh, ph - (ph - fh), ph)
    pw2 = jnp.where(mw, pw - (pw - fw), pw)
    ph3 = jnp.clip(ph2, 0.0, hmax)
    pw3 = jnp.clip(pw2, 0.0, hmax)
    ghlt = 1.0 + (qlth - ph3)
    ghrb = 1.0 - (qrbh - ph3)
    gwlt = 1.0 + (qltw - pw3)
    gwrb = 1.0 - (qrbw - pw3)
    hb = jnp.minimum(qlth, hmax - 1.0)
    wb = jnp.minimum(qltw, hmax - 1.0)
    f32 = jnp.float32
    wh0 = ghlt * (qlth == hb).astype(f32) + ghrb * (qrbh == hb).astype(f32)
    wh1 = (ghlt * (qlth == hb + 1.0).astype(f32)
           + ghrb * (qrbh == hb + 1.0).astype(f32))
    ww0 = gwlt * (qltw == wb).astype(f32) + gwrb * (qrbw == wb).astype(f32)
    ww1 = (gwlt * (qltw == wb + 1.0).astype(f32)
           + gwrb * (qrbw == wb + 1.0).astype(f32))
    idx_ref[...] = (hb * jnp.float32(_HP) + wb).astype(jnp.int32)
    for j, wj in enumerate([wh0 * ww0, wh0 * ww1, wh1 * ww0, wh1 * ww1]):
        wts_ref[:, :, j, :] = wj


def _idx_weights(off):
    nw = _NPIX // _GWIN
    spec_h = pl.BlockSpec((1, nw, _GWIN), lambda k: (2 * k, 0, 0))
    spec_w = pl.BlockSpec((1, nw, _GWIN), lambda k: (2 * k + 1, 0, 0))
    return pl.pallas_call(
        _idx_weights_body,
        grid=(_N,),
        in_specs=[spec_h, spec_w],
        out_specs=[pl.BlockSpec((1, nw, _GWIN), lambda k: (k, 0, 0)),
                   pl.BlockSpec((1, nw, 4, _GWIN), lambda k: (k, 0, 0, 0))],
        out_shape=[jax.ShapeDtypeStruct((_N, nw, _GWIN), jnp.int32),
                   jax.ShapeDtypeStruct((_N, nw, 4, _GWIN), jnp.float32)],
    )(off, off)


_QR = 1600
_QROWS = 51200


def _quads_body(a_ref, b_ref, out_ref):
    cat = jnp.concatenate([a_ref[...], b_ref[...]], axis=0)
    out_ref[...] = jnp.concatenate(
        [cat[0:_QR], cat[1:_QR + 1],
         cat[_HP:_QR + _HP], cat[_HP + 1:_QR + _HP + 1]], axis=1)


def _build_quads(flatp):
    return pl.pallas_call(
        _quads_body,
        grid=(_QROWS // _QR,),
        in_specs=[pl.BlockSpec((_QR, _C), lambda t: (t, 0)),
                  pl.BlockSpec((_QR, _C), lambda t: (t + 1, 0))],
        out_specs=pl.BlockSpec((_QR, 4 * _C), lambda t: (t, 0)),
        out_shape=jax.ShapeDtypeStruct((_QROWS, 4 * _C), jnp.float32),
    )(flatp, flatp)


def _sc_gather_combine(quads, idxf, wtsf):
    mesh = plsc.VectorSubcoreMesh(core_axis_name="c", subcore_axis_name="s")
    cp = pltpu.CompilerParams()
    if "needs_layout_passes" in pltpu.CompilerParams.__dataclass_fields__:
        cp = dataclasses.replace(cp, needs_layout_passes=False)

    @functools.partial(
        pl.kernel,
        out_type=jax.ShapeDtypeStruct((_NSAMP_PAD, _C), jnp.float32),
        mesh=mesh,
        scratch_types=[
            pltpu.VMEM((2, _GWIN), jnp.int32),
            pltpu.VMEM((2, 4, _GWIN), jnp.float32),
            pltpu.VMEM((2, _GWIN, 4 * _C), jnp.float32),
            pltpu.VMEM((_GWIN, _C), jnp.float32),
            pltpu.SemaphoreType.DMA((2,)),
            pltpu.SemaphoreType.DMA((2,)),
            pltpu.SemaphoreType.DMA((2,)),
            pltpu.SemaphoreType.DMA,
        ],
        compiler_params=cp,
    )
    def k(quads_hbm, idx_hbm, wts_hbm, out_hbm,
          idxb, wtsb, gatb, outb, sidx, swts, sgat, sout):
        wid = (jax.lax.axis_index("s") * 2
               + jax.lax.axis_index("c")).astype(jnp.int32)
        base = wid * _WPW

        def idx_copy(w, b):
            return pltpu.make_async_copy(
                idx_hbm.at[w], idxb.at[b], sidx.at[b])

        def wts_copy(w, b):
            return pltpu.make_async_copy(
                wts_hbm.at[w], wtsb.at[b], swts.at[b])

        def gat_copy(b):
            return pltpu.make_async_copy(
                quads_hbm.at[idxb.at[b]], gatb.at[b], sgat.at[b])

        def out_copy(w):
            return pltpu.make_async_copy(
                outb, out_hbm.at[pl.ds(w * _GWIN, _GWIN)], sout)

        for b in range(2):
            idx_copy(base + b, b).start()
            wts_copy(base + b, b).start()
        for b in range(2):
            idx_copy(base + b, b).wait()
            wts_copy(base + b, b).wait()
            gat_copy(b).start()

        last = _WPW // 2 - 1

        @pl.loop(0, _WPW // 2)
        def _(it):
            for b in range(2):
                w = base + it * 2 + b
                gat_copy(b).wait()

                @pl.when(it < last)
                def _():
                    idx_copy(w + 2, b).start()

                @pl.when(w > base)
                def _():
                    out_copy(w - 1).wait()

                @pl.when(it > 0)
                def _():
                    wts_copy(w, b).wait()

                @pl.loop(0, _GWIN)
                def _(i):
                    isplat = jnp.full((_LANES,), i, jnp.int32)
                    wv = [
                        plsc.load_gather(
                            wtsb.at[b],
                            [jnp.full((_LANES,), j, jnp.int32), isplat])
                        for j in range(4)
                    ]
                    for v in range(_C // _LANES):
                        acc = wv[0] * gatb[b, i, pl.ds(v * _LANES, _LANES)]
                        for j in range(1, 4):
                            acc = acc + wv[j] * gatb[
                                b, i, pl.ds(j * _C + v * _LANES, _LANES)]
                        outb[i, pl.ds(v * _LANES, _LANES)] = acc

                @pl.when(it < last)
                def _():
                    wts_copy(w + 2, b).start()

                out_copy(w).start()

                @pl.when(it < last)
                def _():
                    idx_copy(w + 2, b).wait()
                    gat_copy(b).start()

        out_copy(base + _WPW - 1).wait()

    return k(quads, idxf, wtsf)


def _combine_body(*refs):
    gs = refs[:_N]
    w2_ref = refs[_N]
    out_ref = refs[_N + 1]
    acc = jax.lax.dot_general(
        w2_ref[0], gs[0][...], dimension_numbers=(((1,), (1,)), ((), ())),
        preferred_element_type=jnp.float32)
    for k in range(1, _N):
        acc = acc + jax.lax.dot_general(
            w2_ref[k], gs[k][...], dimension_numbers=(((1,), (1,)), ((), ())),
            preferred_element_type=jnp.float32)
    out_ref[...] = acc


def _combine(xoff, w2):
    in_specs = [
        pl.BlockSpec((_T, _C), functools.partial(
            lambda k, t: (k * _NT + t, 0), k))
        for k in range(_N)
    ]
    in_specs.append(pl.BlockSpec((_N, _C, _C), lambda t: (0, 0, 0)))
    return pl.pallas_call(
        _combine_body,
        grid=(_NT,),
        in_specs=in_specs,
        out_specs=pl.BlockSpec((_C, _T), lambda t: (0, t)),
        out_shape=jax.ShapeDtypeStruct((_C, _NPIX), jnp.float32),
    )(*([xoff] * _N), w2)


def kernel(x, offset, W):
    xp = jnp.pad(x[0], ((0, 0), (1, 1), (1, 1)))
    flat = xp.transpose(1, 2, 0).reshape(_HP * _HP, _C)
    flatp = jnp.pad(flat, ((0, _QROWS + _QR - _HP * _HP), (0, 0)))
    quads = _build_quads(flatp)

    off = offset[0].reshape(2 * _N, _NPIX // _GWIN, _GWIN)
    idx, wts = _idx_weights(off)

    nwin = _NSAMP_PAD // _GWIN
    npadw = nwin - _N * (_NPIX // _GWIN)
    idxf = jnp.concatenate(
        [idx.reshape(-1, _GWIN), jnp.zeros((npadw, _GWIN), jnp.int32)])
    wtsf = jnp.concatenate(
        [wts.reshape(-1, 4, _GWIN),
         jnp.zeros((npadw, 4, _GWIN), jnp.float32)])
    xoff = _sc_gather_combine(quads, idxf, wtsf)

    w2 = jnp.transpose(W.reshape(_C, _C, _N), (2, 0, 1))
    out = _combine(xoff, w2)
    return out.reshape(1, _C, _H, _W)
